# Initial kernel scaffold; baseline (speedup 1.0000x reference)
import jax
import jax.numpy as jnp
from jax.experimental import pallas as pl


def kernel(input, covariate, W, a_src, a_dst, W_out, b_out):
    raise NotImplementedError("write your pallas kernel here")


# speedup vs baseline: 2.4199x; 2.4199x over previous
import jax
import jax.numpy as jnp
from jax.experimental import pallas as pl

B, T, N, D_IN = 4, 24, 8192, 32
HID, NH, P, D_OUT = 16, 4, 6, 32
G = T // P
F = NH * HID
NB = 512


def _tb_kernel(x_ref, w2dT_ref, wsrcT_ref, wdstT_ref, woutT_ref, bout_ref,
               out_ref, attn_ref):
    x = x_ref[0]
    xT = x.reshape(T * NB, D_IN).T

    hT = jnp.dot(w2dT_ref[...], xT, preferred_element_type=jnp.float32)
    esT = jnp.dot(wsrcT_ref[...], xT, preferred_element_type=jnp.float32)
    edT = jnp.dot(wdstT_ref[...], xT, preferred_element_type=jnp.float32)

    es4 = esT.reshape(NH, P, G, NB)
    ed4 = edT.reshape(NH, P, G, NB)
    h4 = hT.reshape(F, P, G, NB)

    a_store = [[None] * G for _ in range(G)]
    for i in range(G):
        src = es4[:, :, i]
        e_row = []
        for j in range(G):
            e = src + ed4[:, :, j]
            e_row.append(jnp.where(e >= 0.0, e, 0.2 * e))
        m = jnp.maximum(jnp.maximum(e_row[0], e_row[1]),
                        jnp.maximum(e_row[2], e_row[3]))
        ex = [jnp.exp(e - m) for e in e_row]
        inv = 1.0 / (ex[0] + ex[1] + ex[2] + ex[3])
        for j in range(G):
            a_store[i][j] = ex[j] * inv

    o_rows = []
    for i in range(G):
        acc = None
        for j in range(G):
            arep = jnp.broadcast_to(a_store[i][j][:, None, :, :],
                                    (NH, HID, P, NB)).reshape(F, P, NB)
            c = arep * h4[:, :, j]
            acc = c if acc is None else acc + c
        o_rows.append(acc)
    oT = jnp.stack(o_rows, axis=2).reshape(F, T * NB)

    zT = jnp.dot(woutT_ref[...], oT, preferred_element_type=jnp.float32)
    zT = zT + bout_ref[...]
    zT = jnp.where(zT > 0.0, zT, jnp.exp(zT) - 1.0)
    outT = xT + zT
    out_ref[0] = outT.T.reshape(T, NB, D_IN)

    a_ij = jnp.stack([jnp.stack(a_store[i], axis=2) for i in range(G)],
                     axis=2)
    attn_ref[...] = a_ij.reshape(NH * P * G * G, NB).T


def kernel(input, covariate, W, a_src, a_dst, W_out, b_out):
    del covariate
    w2dT = jnp.transpose(W, (0, 2, 1)).reshape(F, D_IN)
    wsrcT = jnp.einsum('ndh,nh->nd', W, a_src)
    wdstT = jnp.einsum('ndh,nh->nd', W, a_dst)
    woutT = W_out.T
    bout = b_out.reshape(D_OUT, 1)

    nblk = N // NB
    out, attn2 = pl.pallas_call(
        _tb_kernel,
        grid=(B, nblk),
        in_specs=[
            pl.BlockSpec((1, T, NB, D_IN), lambda b, k: (b, 0, k, 0)),
            pl.BlockSpec((F, D_IN), lambda b, k: (0, 0)),
            pl.BlockSpec((NH, D_IN), lambda b, k: (0, 0)),
            pl.BlockSpec((NH, D_IN), lambda b, k: (0, 0)),
            pl.BlockSpec((D_OUT, F), lambda b, k: (0, 0)),
            pl.BlockSpec((D_OUT, 1), lambda b, k: (0, 0)),
        ],
        out_specs=[
            pl.BlockSpec((1, T, NB, D_IN), lambda b, k: (b, 0, k, 0)),
            pl.BlockSpec((NB, NH * P * G * G), lambda b, k: (b * nblk + k, 0)),
        ],
        out_shape=[
            jax.ShapeDtypeStruct((B, T, N, D_IN), jnp.float32),
            jax.ShapeDtypeStruct((B * N, NH * P * G * G), jnp.float32),
        ],
    )(input, w2dT, wsrcT, wdstT, woutT, bout)

    return (out, attn2.reshape(B * N, NH, P, G, G))


# speedup vs baseline: 3.2133x; 1.3279x over previous
import jax
import jax.numpy as jnp
from jax.experimental import pallas as pl

B, T, N, D_IN = 4, 24, 8192, 32
HID, NH, P, D_OUT = 16, 4, 6, 32
G = T // P
F = NH * HID
NB = 512
AC = NH * P * G * G


def _tb_kernel(x_ref, w2dT_ref, wsrcT_ref, wdstT_ref, woutT_ref, bout_ref,
               rep_ref, perm_ref, out_ref, attn_ref):
    x = x_ref[0]
    xT = x.reshape(T * NB, D_IN).T

    hT = jnp.dot(w2dT_ref[...], xT, preferred_element_type=jnp.float32)
    esT = jnp.dot(wsrcT_ref[...], xT, preferred_element_type=jnp.float32)
    edT = jnp.dot(wdstT_ref[...], xT, preferred_element_type=jnp.float32)
    rep = rep_ref[...]

    o_chunks = []
    a_pieces = []
    for p in range(P):
        base = p * G * NB
        hslab = hT[:, base:base + G * NB]
        src = [esT[:, base + i * NB: base + (i + 1) * NB] for i in range(G)]
        dst = [edT[:, base + j * NB: base + (j + 1) * NB] for j in range(G)]
        for i in range(G):
            e_row = []
            for j in range(G):
                e = src[i] + dst[j]
                e_row.append(jnp.where(e >= 0.0, e, 0.2 * e))
            m = jnp.maximum(jnp.maximum(e_row[0], e_row[1]),
                            jnp.maximum(e_row[2], e_row[3]))
            ex = [jnp.exp(e - m) for e in e_row]
            inv = 1.0 / (ex[0] + ex[1] + ex[2] + ex[3])
            a_row = [exj * inv for exj in ex]
            a_pieces.extend(a_row)
            a_pi = jnp.concatenate(a_row, axis=1)
            arep = jnp.dot(rep, a_pi, preferred_element_type=jnp.float32)
            c = arep * hslab
            o_chunks.append(c[:, 0:NB] + c[:, NB:2 * NB]
                            + c[:, 2 * NB:3 * NB] + c[:, 3 * NB:4 * NB])

    oT = jnp.concatenate(o_chunks, axis=1)
    zT = jnp.dot(woutT_ref[...], oT, preferred_element_type=jnp.float32)
    zT = zT + bout_ref[...]
    zT = jnp.where(zT > 0.0, zT, jnp.exp(zT) - 1.0)
    outT = xT + zT
    out_ref[0] = outT.T.reshape(T, NB, D_IN)

    a0 = jnp.concatenate(a_pieces, axis=0)
    attn_ref[...] = jnp.dot(a0.T, perm_ref[...],
                            preferred_element_type=jnp.float32)


def kernel(input, covariate, W, a_src, a_dst, W_out, b_out):
    del covariate
    w2dT = jnp.transpose(W, (0, 2, 1)).reshape(F, D_IN)
    wsrcT = jnp.einsum('ndh,nh->nd', W, a_src)
    wdstT = jnp.einsum('ndh,nh->nd', W, a_dst)
    woutT = W_out.T
    bout = b_out.reshape(D_OUT, 1)
    rep = (jnp.arange(F)[:, None] // HID
           == jnp.arange(NH)[None, :]).astype(jnp.float32)
    d = jnp.arange(AC)
    s_of_d = (d % (P * G * G)) * NH + d // (P * G * G)
    perm = (jnp.arange(AC)[:, None] == s_of_d[None, :]).astype(jnp.float32)

    nblk = N // NB
    out, attn2 = pl.pallas_call(
        _tb_kernel,
        grid=(B, nblk),
        in_specs=[
            pl.BlockSpec((1, T, NB, D_IN), lambda b, k: (b, 0, k, 0)),
            pl.BlockSpec((F, D_IN), lambda b, k: (0, 0)),
            pl.BlockSpec((NH, D_IN), lambda b, k: (0, 0)),
            pl.BlockSpec((NH, D_IN), lambda b, k: (0, 0)),
            pl.BlockSpec((D_OUT, F), lambda b, k: (0, 0)),
            pl.BlockSpec((D_OUT, 1), lambda b, k: (0, 0)),
            pl.BlockSpec((F, NH), lambda b, k: (0, 0)),
            pl.BlockSpec((AC, AC), lambda b, k: (0, 0)),
        ],
        out_specs=[
            pl.BlockSpec((1, T, NB, D_IN), lambda b, k: (b, 0, k, 0)),
            pl.BlockSpec((NB, AC), lambda b, k: (b * nblk + k, 0)),
        ],
        out_shape=[
            jax.ShapeDtypeStruct((B, T, N, D_IN), jnp.float32),
            jax.ShapeDtypeStruct((B * N, AC), jnp.float32),
        ],
    )(input, w2dT, wsrcT, wdstT, woutT, bout, rep, perm)

    return (out, attn2.reshape(B * N, NH, P, G, G))
